# Initial kernel scaffold; baseline (speedup 1.0000x reference)
import jax
import jax.numpy as jnp
from jax.experimental import pallas as pl


def kernel(x, w0, sb0, w1, sb1, w2, sb2, w3, sb3, w4, sb4, w5, sb5, w6, sb6, w7, sb7):
    raise NotImplementedError("write your pallas kernel here")


# speedup vs baseline: 2.2580x; 2.2580x over previous
import jax
import jax.numpy as jnp
from jax.experimental import pallas as pl
from jax.experimental.pallas import tpu as pltpu

_SLOPE = 0.2


def _leaky(y):
    return jnp.maximum(y, _SLOPE * y)


def _l0_body(p_ref, w_ref, sb_ref, o_ref):
    B, M, K = p_ref.shape
    x = p_ref[...].reshape(B * M, K)
    y = jnp.dot(x, w_ref[...], preferred_element_type=jnp.float32)
    y = _leaky(y * sb_ref[0:1, :] + sb_ref[1:2, :])
    o_ref[...] = y.reshape(B, M, o_ref.shape[-1]).astype(o_ref.dtype)


def _conv0(x_nhwc, w0, sb0):
    N = x_nhwc.shape[0]
    xp = jnp.pad(x_nhwc, ((0, 0), (1, 1), (1, 1), (0, 0)))
    pats = [xp[:, kh:kh + 64:2, kw:kw + 64:2, :]
            for kh in range(4) for kw in range(4)]
    patches = jnp.concatenate(pats, axis=-1).reshape(N, 1024, 48)
    w = w0[:, :3, :].reshape(48, 128)
    B = min(8, N)
    out = pl.pallas_call(
        _l0_body,
        out_shape=jax.ShapeDtypeStruct((N, 1024, 128), jnp.bfloat16),
        grid=(N // B,),
        in_specs=[pl.BlockSpec((B, 1024, 48), lambda n: (n, 0, 0)),
                  pl.BlockSpec((48, 128), lambda n: (0, 0)),
                  pl.BlockSpec((2, 128), lambda n: (0, 0))],
        out_specs=pl.BlockSpec((B, 1024, 128), lambda n: (n, 0, 0)),
        compiler_params=pltpu.CompilerParams(
            dimension_semantics=("parallel",)),
    )(patches, w, sb0)
    return out.reshape(N, 32, 32, 128)


def _make_s2_body(B, Ho, Wo):
    def body(p00, p01, p10, p11, w_ref, sb_ref, o_ref):
        planes = ((p00, p01), (p10, p11))
        cin = p00.shape[-1]
        acc = None
        for kh in range(3):
            for kw in range(3):
                pr = planes[kh % 2][kw % 2]
                dh, dw = kh // 2, kw // 2
                slab = pr[:, dh:dh + Ho, dw:dw + Wo, :]
                slab = slab.reshape(B * Ho * Wo, cin)
                d = jnp.dot(slab, w_ref[kh * 3 + kw],
                            preferred_element_type=jnp.float32)
                acc = d if acc is None else acc + d
        y = _leaky(acc * sb_ref[0:1, :] + sb_ref[1:2, :])
        o_ref[...] = y.reshape(B, Ho * Wo, o_ref.shape[-1]).astype(o_ref.dtype)
    return body


def _conv_s2(x, w, sb, B):
    N, H, _, C = x.shape
    B = min(B, N)
    Ho = H // 2
    xp = jnp.pad(x, ((0, 0), (1, 1), (1, 1), (0, 0)))
    planes = [xp[:, rp::2, cp::2, :] for rp in (0, 1) for cp in (0, 1)]
    cout = w.shape[-1]
    out = pl.pallas_call(
        _make_s2_body(B, Ho, Ho),
        out_shape=jax.ShapeDtypeStruct((N, Ho * Ho, cout), jnp.bfloat16),
        grid=(N // B,),
        in_specs=[pl.BlockSpec((B, Ho + 1, Ho + 1, C),
                               lambda n: (n, 0, 0, 0))] * 4 + [
            pl.BlockSpec(w.shape, lambda n: (0, 0, 0)),
            pl.BlockSpec(sb.shape, lambda n: (0, 0))],
        out_specs=pl.BlockSpec((B, Ho * Ho, cout), lambda n: (n, 0, 0)),
        compiler_params=pltpu.CompilerParams(
            dimension_semantics=("parallel",)),
    )(*planes, w, sb)
    return out


def _dense_w(w_taps, cin_p, cout_p):
    wb = jnp.zeros((4 * cin_p, 4 * cout_p), jnp.bfloat16)
    for ho in range(2):
        for wo in range(2):
            p = ho * 2 + wo
            for hi in range(2):
                for wi in range(2):
                    q = hi * 2 + wi
                    t = (hi - ho + 1) * 3 + (wi - wo + 1)
                    wb = wb.at[q * cin_p:(q + 1) * cin_p,
                               p * cout_p:(p + 1) * cout_p].set(w_taps[t])
    return wb


def _tail_a_body(y3_ref, w4_ref, sb4_ref, w5_ref, sb5_ref, o_ref):
    acc5 = None
    for ho in range(2):
        for wo in range(2):
            acc = None
            for hi in range(4):
                kh = hi - 2 * ho + 1
                if kh < 0 or kh > 2:
                    continue
                for wi in range(4):
                    kw = wi - 2 * wo + 1
                    if kw < 0 or kw > 2:
                        continue
                    d = jnp.dot(y3_ref[:, hi * 4 + wi, :],
                                w4_ref[kh * 3 + kw],
                                preferred_element_type=jnp.float32)
                    acc = d if acc is None else acc + d
            y4p = _leaky(acc * sb4_ref[0:1, :] + sb4_ref[1:2, :])
            p = ho * 2 + wo
            d5 = jnp.dot(y4p.astype(jnp.bfloat16),
                         w5_ref[p * 512:(p + 1) * 512, :],
                         preferred_element_type=jnp.float32)
            acc5 = d5 if acc5 is None else acc5 + d5
    y5 = _leaky(acc5 * sb5_ref[0:1, :] + sb5_ref[1:2, :])
    o_ref[...] = y5.astype(o_ref.dtype)


def _tail_b_body(x_ref, w6_ref, sb6_ref, w7_ref, sb7_ref, o_ref):
    y6 = jnp.dot(x_ref[...], w6_ref[...], preferred_element_type=jnp.float32)
    y6 = _leaky(y6 * sb6_ref[0:1, :] + sb6_ref[1:2, :]).astype(jnp.bfloat16)
    y7 = jnp.dot(y6, w7_ref[...], preferred_element_type=jnp.float32)
    y7 = _leaky(y7 * sb7_ref[0:1, :] + sb7_ref[1:2, :])
    o_ref[...] = y7


def _tail(y3, w4, sb4, w5, sb5, w6, sb6, w7, sb7):
    N = y3.shape[0]
    w5b = _dense_w(w5, 512, 1024)
    w6b = _dense_w(w6, 1024, 512)
    w7b = _dense_w(w7, 512, 128)
    sb5t = jnp.tile(sb5, (1, 4))
    sb6t = jnp.tile(sb6, (1, 4))
    sb7t = jnp.tile(sb7, (1, 4))
    B = min(64, N)
    y5 = pl.pallas_call(
        _tail_a_body,
        out_shape=jax.ShapeDtypeStruct((N, 4096), jnp.bfloat16),
        grid=(N // B,),
        in_specs=[pl.BlockSpec((B, 16, 256), lambda n: (n, 0, 0)),
                  pl.BlockSpec(w4.shape, lambda n: (0, 0, 0)),
                  pl.BlockSpec(sb4.shape, lambda n: (0, 0)),
                  pl.BlockSpec((2048, 4096), lambda n: (0, 0)),
                  pl.BlockSpec((2, 4096), lambda n: (0, 0))],
        out_specs=pl.BlockSpec((B, 4096), lambda n: (n, 0)),
        compiler_params=pltpu.CompilerParams(
            dimension_semantics=("parallel",)),
    )(y3, w4, sb4, w5b, sb5t)
    y7 = pl.pallas_call(
        _tail_b_body,
        out_shape=jax.ShapeDtypeStruct((N, 512), jnp.float32),
        grid=(N // B,),
        in_specs=[pl.BlockSpec((B, 4096), lambda n: (n, 0)),
                  pl.BlockSpec((4096, 2048), lambda n: (0, 0)),
                  pl.BlockSpec((2, 2048), lambda n: (0, 0)),
                  pl.BlockSpec((2048, 512), lambda n: (0, 0)),
                  pl.BlockSpec((2, 512), lambda n: (0, 0))],
        out_specs=pl.BlockSpec((B, 512), lambda n: (n, 0)),
        compiler_params=pltpu.CompilerParams(
            dimension_semantics=("parallel",)),
    )(y5, w6b, sb6t, w7b, sb7t)
    return y7


def kernel(x, w0, sb0, w1, sb1, w2, sb2, w3, sb3,
           w4, sb4, w5, sb5, w6, sb6, w7, sb7):
    N = x.shape[0]
    xh = jnp.transpose(x, (0, 2, 3, 1)).astype(jnp.bfloat16)
    y = _conv0(xh, w0, sb0)
    y = _conv_s2(y, w1, sb1, B=16).reshape(N, 16, 16, 128)
    y = _conv_s2(y, w2, sb2, B=32).reshape(N, 8, 8, 128)
    y3 = _conv_s2(y, w3, sb3, B=64)
    y7 = _tail(y3, w4, sb4, w5, sb5, w6, sb6, w7, sb7)
    out = y7.reshape(N, 2, 2, 128)[..., :35]
    return out.reshape(N, 2, 2, 5, 7)


# speedup vs baseline: 46.7933x; 20.7229x over previous
import functools

import jax
import jax.numpy as jnp
from jax.experimental import pallas as pl
from jax.experimental.pallas import tpu as pltpu

_SLOPE = 0.2


def _leaky(y):
    return jnp.maximum(y, _SLOPE * y)


def _l0_body(B, x_ref, w_ref, sb_ref, o_ref):
    z = x_ref[...].reshape(B, 33, 2, 33, 6)
    slabs = [z[:, kh // 2:kh // 2 + 32, kh % 2, dw:dw + 32, :]
             for kh in range(4) for dw in range(2)]
    pat = jnp.concatenate(slabs, axis=-1)
    x = pat.reshape(B * 1024, 48)
    y = jnp.dot(x, w_ref[...], preferred_element_type=jnp.float32)
    y = _leaky(y * sb_ref[0:1, :] + sb_ref[1:2, :])
    o_ref[...] = y.reshape(B, 1024, o_ref.shape[-1]).astype(o_ref.dtype)


def _conv0(x_nhwc, w0, sb0):
    N = x_nhwc.shape[0]
    xp = jnp.pad(x_nhwc, ((0, 0), (1, 1), (1, 1), (0, 0)))
    xq = xp.reshape(N, 66, 33, 6)
    w = w0[:, :3, :].reshape(48, 128)
    B = min(8, N)
    out = pl.pallas_call(
        functools.partial(_l0_body, B),
        out_shape=jax.ShapeDtypeStruct((N, 1024, 128), jnp.bfloat16),
        grid=(N // B,),
        in_specs=[pl.BlockSpec((B, 66, 33, 6), lambda n: (n, 0, 0, 0)),
                  pl.BlockSpec((48, 128), lambda n: (0, 0)),
                  pl.BlockSpec((2, 128), lambda n: (0, 0))],
        out_specs=pl.BlockSpec((B, 1024, 128), lambda n: (n, 0, 0)),
        compiler_params=pltpu.CompilerParams(
            dimension_semantics=("parallel",)),
    )(xq, w, sb0)
    return out.reshape(N, 32, 32, 128)


def _make_s2_body(B, Ho, Wo, C):
    def body(x_ref, w01_ref, w2_ref, sb_ref, o_ref):
        z = x_ref[...].reshape(B, Ho + 1, 2, Wo + 1, 2 * C)
        acc = None
        for kh in range(3):
            rows = z[:, kh // 2:kh // 2 + Ho, kh % 2]
            s01 = rows[:, :, 0:Wo, :].reshape(B * Ho * Wo, 2 * C)
            d = jnp.dot(s01, w01_ref[kh], preferred_element_type=jnp.float32)
            acc = d if acc is None else acc + d
            s2 = rows[:, :, 1:1 + Wo, 0:C].reshape(B * Ho * Wo, C)
            acc = acc + jnp.dot(s2, w2_ref[kh],
                                preferred_element_type=jnp.float32)
        y = _leaky(acc * sb_ref[0:1, :] + sb_ref[1:2, :])
        o_ref[...] = y.reshape(B, Ho * Wo, o_ref.shape[-1]).astype(o_ref.dtype)
    return body


def _conv_s2(x, w, sb, B, creal):
    N, H, _, C = x.shape
    B = min(B, N)
    Ho = H // 2
    xp = jnp.pad(x[..., :creal], ((0, 0), (1, 1), (1, 1), (0, 0)))
    xq = xp.reshape(N, 2 * Ho + 2, Ho + 1, 2 * creal)
    cout = w.shape[-1]
    wk = w.reshape(3, 3, C, cout)[:, :, :creal, :]
    w01 = wk[:, 0:2].reshape(3, 2 * creal, cout)
    w2 = wk[:, 2]
    out = pl.pallas_call(
        _make_s2_body(B, Ho, Ho, creal),
        out_shape=jax.ShapeDtypeStruct((N, Ho * Ho, cout), jnp.bfloat16),
        grid=(N // B,),
        in_specs=[pl.BlockSpec((B, 2 * Ho + 2, Ho + 1, 2 * creal),
                               lambda n: (n, 0, 0, 0)),
                  pl.BlockSpec(w01.shape, lambda n: (0, 0, 0)),
                  pl.BlockSpec(w2.shape, lambda n: (0, 0, 0)),
                  pl.BlockSpec(sb.shape, lambda n: (0, 0))],
        out_specs=pl.BlockSpec((B, Ho * Ho, cout), lambda n: (n, 0, 0)),
        compiler_params=pltpu.CompilerParams(
            dimension_semantics=("parallel",)),
    )(xq, w01, w2, sb)
    return out


def _tail_body(y3_ref, w4_ref, sb4_ref, w5_ref, sb5_ref, w6_ref, sb6_ref,
               w7_ref, sb7_ref, o_ref):
    def ep(acc, sb_ref):
        return _leaky(acc * sb_ref[0:1, :] + sb_ref[1:2, :])

    y4 = []
    for ho in range(2):
        for wo in range(2):
            acc = None
            for hi in range(4):
                kh = hi - 2 * ho + 1
                if kh < 0 or kh > 2:
                    continue
                for wi in range(4):
                    kw = wi - 2 * wo + 1
                    if kw < 0 or kw > 2:
                        continue
                    d = jnp.dot(y3_ref[:, hi * 4 + wi, :],
                                w4_ref[kh * 3 + kw],
                                preferred_element_type=jnp.float32)
                    acc = d if acc is None else acc + d
            y4.append(ep(acc, sb4_ref).astype(jnp.bfloat16))

    def conv2x2(xs, w_ref, sb_ref):
        ys = []
        for ho in range(2):
            for wo in range(2):
                acc = None
                for hi in range(2):
                    for wi in range(2):
                        t = (hi - ho + 1) * 3 + (wi - wo + 1)
                        d = jnp.dot(xs[hi * 2 + wi], w_ref[t],
                                    preferred_element_type=jnp.float32)
                        acc = d if acc is None else acc + d
                ys.append(ep(acc, sb_ref))
        return ys

    y5 = [y.astype(jnp.bfloat16) for y in conv2x2(y4, w5_ref, sb5_ref)]
    y6 = [y.astype(jnp.bfloat16) for y in conv2x2(y5, w6_ref, sb6_ref)]
    y7 = conv2x2(y6, w7_ref, sb7_ref)
    c7 = o_ref.shape[-1] // 4
    for p in range(4):
        o_ref[:, p * c7:(p + 1) * c7] = y7[p]


def _tail(y3, w4, sb4, w5, sb5, w6, sb6, w7, sb7):
    N = y3.shape[0]
    B = min(64, N)
    return pl.pallas_call(
        _tail_body,
        out_shape=jax.ShapeDtypeStruct((N, 512), jnp.float32),
        grid=(N // B,),
        in_specs=[pl.BlockSpec((B, 16, 256), lambda n: (n, 0, 0)),
                  pl.BlockSpec(w4.shape, lambda n: (0, 0, 0)),
                  pl.BlockSpec(sb4.shape, lambda n: (0, 0)),
                  pl.BlockSpec(w5.shape, lambda n: (0, 0, 0)),
                  pl.BlockSpec(sb5.shape, lambda n: (0, 0)),
                  pl.BlockSpec(w6.shape, lambda n: (0, 0, 0)),
                  pl.BlockSpec(sb6.shape, lambda n: (0, 0)),
                  pl.BlockSpec(w7.shape, lambda n: (0, 0, 0)),
                  pl.BlockSpec(sb7.shape, lambda n: (0, 0))],
        out_specs=pl.BlockSpec((B, 512), lambda n: (n, 0)),
        compiler_params=pltpu.CompilerParams(
            dimension_semantics=("parallel",)),
    )(y3, w4, sb4, w5, sb5, w6, sb6, w7, sb7)


def kernel(x, w0, sb0, w1, sb1, w2, sb2, w3, sb3,
           w4, sb4, w5, sb5, w6, sb6, w7, sb7):
    N = x.shape[0]
    xh = jnp.transpose(x, (0, 2, 3, 1)).astype(jnp.bfloat16)
    y = _conv0(xh, w0, sb0)
    y = _conv_s2(y, w1, sb1, B=16, creal=64).reshape(N, 16, 16, 128)
    y = _conv_s2(y, w2, sb2, B=32, creal=64).reshape(N, 8, 8, 128)
    y3 = _conv_s2(y, w3, sb3, B=64, creal=128)
    y7 = _tail(y3, w4, sb4, w5, sb5, w6, sb6, w7, sb7)
    out = y7.reshape(N, 2, 2, 128)[..., :35]
    return out.reshape(N, 2, 2, 5, 7)
